# baseline (speedup 1.0000x reference)
import functools
import math

import jax
import jax.numpy as jnp
from jax import lax
from jax.experimental import pallas as pl
from jax.experimental.pallas import tpu as pltpu
from jax.experimental.pallas import tpu_sc as plsc

D_MODEL = 64
SCALE = math.sqrt(D_MODEL)


@functools.cache
def _build(B, NC, NS):
    NW = NC * NS
    BPW = B // NW
    CH = 128
    NCHUNK = BPW // CH

    mesh = plsc.VectorSubcoreMesh(core_axis_name="c", subcore_axis_name="s")

    @functools.partial(
        pl.kernel,
        mesh=mesh,
        compiler_params=pltpu.CompilerParams(use_tc_tiling_on_sc=False),
        out_type=jax.ShapeDtypeStruct((B, D_MODEL), jnp.float32),
        scratch_types=[
            pltpu.VMEM((NCHUNK, CH), jnp.int32),
            pltpu.VMEM((CH, D_MODEL), jnp.float32),
            pltpu.SemaphoreType.DMA,
        ],
    )
    def k(x_hbm, table_hbm, out_hbm, idx_v, rows_v, sem):
        wid = lax.axis_index("s") * NC + lax.axis_index("c")
        base = wid * BPW
        pltpu.sync_copy(x_hbm.at[pl.ds(wid * NCHUNK, NCHUNK)], idx_v)

        def chunk_body(c, carry):
            pltpu.async_copy(table_hbm.at[idx_v.at[c]], rows_v, sem).wait()

            def scale_row(r, carry2):
                for j in range(D_MODEL // 16):
                    sl = pl.ds(j * 16, 16)
                    rows_v[r, sl] = rows_v[r, sl] * SCALE
                return carry2

            lax.fori_loop(0, CH, scale_row, 0)
            pltpu.sync_copy(rows_v, out_hbm.at[pl.ds(base + c * CH, CH)])
            return carry

        lax.fori_loop(0, NCHUNK, chunk_body, 0)

    return k


def kernel(x, table):
    S0, S1 = x.shape
    B = S0 * S1
    info = plsc.get_sparse_core_info()
    NC, NS = info.num_cores, info.num_subcores
    CH = 128
    x2 = x.reshape(B // CH, CH).astype(jnp.int32)
    out = _build(B, NC, NS)(x2, table)
    return out.reshape(S0, S1, D_MODEL)


# speedup vs baseline: 1.2064x; 1.2064x over previous
import functools
import math

import jax
import jax.numpy as jnp
from jax import lax
from jax.experimental import pallas as pl
from jax.experimental.pallas import tpu as pltpu
from jax.experimental.pallas import tpu_sc as plsc

D_MODEL = 64
SCALE = math.sqrt(D_MODEL)
CH = 128
RING = 4
AHEAD = 2


@functools.cache
def _build(B, NC, NS):
    NW = NC * NS
    BPW = B // NW
    NCHUNK = BPW // CH
    NGROUP = NCHUNK // RING

    mesh = plsc.VectorSubcoreMesh(core_axis_name="c", subcore_axis_name="s")

    @functools.partial(
        pl.kernel,
        mesh=mesh,
        compiler_params=pltpu.CompilerParams(use_tc_tiling_on_sc=False),
        out_type=jax.ShapeDtypeStruct((B, D_MODEL), jnp.float32),
        scratch_types=[
            pltpu.VMEM((NCHUNK, CH), jnp.int32),
            pltpu.VMEM((RING, CH, D_MODEL), jnp.float32),
            pltpu.SemaphoreType.DMA((RING,)),
            pltpu.SemaphoreType.DMA((RING,)),
        ],
    )
    def k(x_hbm, table_hbm, out_hbm, idx_v, rows_v, gsem, osem):
        wid = lax.axis_index("s") * NC + lax.axis_index("c")
        base = wid * BPW
        pltpu.sync_copy(x_hbm.at[pl.ds(wid * NCHUNK, NCHUNK)], idx_v)

        def gather_start(c, slot):
            pltpu.make_async_copy(
                table_hbm.at[idx_v.at[c]], rows_v.at[slot], gsem.at[slot]
            ).start()

        def gather_wait(c, slot):
            pltpu.make_async_copy(
                table_hbm.at[idx_v.at[c]], rows_v.at[slot], gsem.at[slot]
            ).wait()

        def out_copy(c, slot):
            return pltpu.make_async_copy(
                rows_v.at[slot], out_hbm.at[pl.ds(base + c * CH, CH)],
                osem.at[slot],
            )

        for b in range(AHEAD):
            gather_start(b, b)

        def group_body(g, carry):
            for b in range(RING):
                c = g * RING + b
                c2 = c + AHEAD
                slot2 = (b + AHEAD) % RING

                @pl.when(c2 < NCHUNK)
                def _issue():
                    @pl.when(c2 >= RING)
                    def _drain():
                        out_copy(c2 - RING, slot2).wait()

                    gather_start(c2, slot2)

                gather_wait(c, b)

                def scale_row(r, carry2):
                    for j in range(D_MODEL // 16):
                        sl = pl.ds(j * 16, 16)
                        rows_v[b, r, sl] = rows_v[b, r, sl] * SCALE
                    return carry2

                lax.fori_loop(0, CH, scale_row, 0)
                out_copy(c, b).start()
            return carry

        lax.fori_loop(0, NGROUP, group_body, 0)

        for b in range(RING):
            out_copy(NCHUNK - RING + b, b).wait()

    return k


def kernel(x, table):
    S0, S1 = x.shape
    B = S0 * S1
    info = plsc.get_sparse_core_info()
    NC, NS = info.num_cores, info.num_subcores
    x2 = x.reshape(B // CH, CH).astype(jnp.int32)
    out = _build(B, NC, NS)(x2, table)
    return out.reshape(S0, S1, D_MODEL)
